# Initial kernel scaffold; baseline (speedup 1.0000x reference)
import jax
import jax.numpy as jnp
from jax.experimental import pallas as pl


def kernel(x, edge_index, edge_weight, W1, b1, W2, b2, Wl, bl):
    raise NotImplementedError("write your pallas kernel here")


# speedup vs baseline: 31.2712x; 31.2712x over previous
import functools
import jax
import jax.numpy as jnp
import numpy as np
from jax import lax
from jax.experimental import pallas as pl
from jax.experimental.pallas import tpu as pltpu
from jax.experimental.pallas import tpu_sc as plsc

NC = 2
NS = 16
LANES = 16
C = 128


def _mesh():
    return plsc.VectorSubcoreMesh(
        core_axis_name="c", subcore_axis_name="s", num_cores=NC, num_subcores=NS
    )


def _deg_kernel(n_nodes, chunks_per_worker, dst_hbm, w_hbm, out_hbm,
                dst_v, w_v, acc_sh, zbuf):
    cid = lax.axis_index("c")
    sid = lax.axis_index("s")
    wid = sid * NC + cid
    start = wid * chunks_per_worker

    @pl.when(sid == 0)
    def _():
        def zero_body(i, _):
            zbuf[pl.ds(i * LANES, LANES)] = jnp.zeros((LANES,), jnp.float32)
            return _
        lax.fori_loop(0, n_nodes // LANES, zero_body, None)
        pltpu.sync_copy(zbuf, acc_sh)

    plsc.subcore_barrier()

    pltpu.sync_copy(dst_hbm.at[pl.ds(start, chunks_per_worker)], dst_v)
    pltpu.sync_copy(w_hbm.at[pl.ds(start, chunks_per_worker)], w_v)

    def chunk_body(j, _):
        pltpu.sync_copy(w_v.at[j], acc_sh.at[dst_v.at[j]], add=True)
        return _
    lax.fori_loop(0, chunks_per_worker, chunk_body, None)

    plsc.subcore_barrier()

    @pl.when(sid == 0)
    def _():
        pltpu.sync_copy(acc_sh, out_hbm.at[cid])


def _agg_kernel(n_nodes, chunks_per_worker, src_hbm, dst_hbm, w_hbm, y_hbm,
                out_hbm, src_v, dst_v, w_v, rows, acc_sh, gsem):
    cid = lax.axis_index("c")
    sid = lax.axis_index("s")
    wid = sid * NC + cid
    start = wid * chunks_per_worker

    @pl.when(sid == 0)
    def _():
        pltpu.sync_copy(y_hbm, acc_sh)

    plsc.subcore_barrier()

    pltpu.sync_copy(src_hbm.at[pl.ds(start, chunks_per_worker)], src_v)
    pltpu.sync_copy(dst_hbm.at[pl.ds(start, chunks_per_worker)], dst_v)
    pltpu.sync_copy(w_hbm.at[pl.ds(start * C, chunks_per_worker * C)], w_v)

    def chunk_body(j, _):
        pltpu.async_copy(y_hbm.at[src_v.at[j]], rows, gsem).wait()

        base = j * C

        def scale_body(g, _):
            gbase = g * LANES
            wv = w_v[pl.ds(base + gbase, LANES)]
            for t in range(LANES):
                k = gbase + t
                wk = jnp.full((LANES,), wv[t])
                rows[k, pl.ds(0, LANES)] = rows[k, pl.ds(0, LANES)] * wk
                rows[k, pl.ds(LANES, LANES)] = rows[k, pl.ds(LANES, LANES)] * wk
            return _
        lax.fori_loop(0, C // LANES, scale_body, None)

        pltpu.sync_copy(rows, acc_sh.at[dst_v.at[j]], add=True)
        return _
    lax.fori_loop(0, chunks_per_worker, chunk_body, None)

    plsc.subcore_barrier()

    @pl.when(sid == 0)
    def _():
        pltpu.sync_copy(acc_sh, out_hbm.at[cid])


def _tc_k1(x_ref, w1_ref, degp_ref, y_ref, dinv_ref):
    deg = degp_ref[0, :] + degp_ref[1, :] + 1.0
    dinv = lax.rsqrt(deg)
    xw = jnp.dot(x_ref[...], w1_ref[...], preferred_element_type=jnp.float32)
    y_ref[...] = xw * dinv[:, None]
    dinv_ref[...] = dinv


def _tc_k2(accp_ref, y1_ref, dinv_ref, b1_ref, w2_ref, y2_ref):
    y1 = y1_ref[...]
    dinv = dinv_ref[...]
    t = accp_ref[0] + accp_ref[1] - y1
    h1 = jnp.maximum(dinv[:, None] * t + b1_ref[...][None, :], 0.0)
    xw2 = jnp.dot(h1, w2_ref[...], preferred_element_type=jnp.float32)
    y2_ref[...] = xw2 * dinv[:, None]


def _tc_k3(accp_ref, y2_ref, dinv_ref, b2_ref, wl_ref, bl_ref, out_ref):
    y2 = y2_ref[...]
    dinv = dinv_ref[...]
    t = accp_ref[0] + accp_ref[1] - y2
    h2 = jnp.maximum(dinv[:, None] * t + b2_ref[...][None, :], 0.0)
    res = jnp.dot(h2, wl_ref[...], preferred_element_type=jnp.float32)
    out_ref[...] = res[:, 0] + bl_ref[0]


@jax.jit
def kernel(x, edge_index, edge_weight, W1, b1, W2, b2, Wl, bl):
    n_nodes, d_in = x.shape
    h = W1.shape[1]
    e = edge_index.shape[1]

    n_workers = NC * NS
    chunks = -(-e // C)
    cpw = -(-chunks // n_workers)
    cpw = -(-cpw // 8) * 8
    e_pad = n_workers * cpw * C

    pad = e_pad - e
    src = edge_index[0]
    dst = edge_index[1]
    w = edge_weight
    if pad:
        fill = (jnp.arange(pad, dtype=jnp.int32) * 97) % n_nodes
        src = jnp.concatenate([src, fill])
        dst = jnp.concatenate([dst, fill])
        w = jnp.concatenate([w, jnp.zeros((pad,), jnp.float32)])
    src2d = src.reshape(-1, C)
    dst2d = dst.reshape(-1, C)
    w2d = w.reshape(-1, C)

    mesh = _mesh()

    deg_fn = pl.kernel(
        functools.partial(_deg_kernel, n_nodes, cpw),
        out_type=jax.ShapeDtypeStruct((NC, n_nodes), jnp.float32),
        mesh=mesh,
        scratch_types=[
            pltpu.MemorySpace.VMEM((cpw, C), jnp.int32),
            pltpu.MemorySpace.VMEM((cpw, C), jnp.float32),
            pltpu.MemorySpace.VMEM_SHARED((n_nodes,), jnp.float32),
            pltpu.MemorySpace.VMEM((n_nodes,), jnp.float32),
        ],
    )
    degp = deg_fn(dst2d, w2d)

    agg_fn = pl.kernel(
        functools.partial(_agg_kernel, n_nodes, cpw),
        out_type=jax.ShapeDtypeStruct((NC, n_nodes, h), jnp.float32),
        mesh=mesh,
        compiler_params=pltpu.CompilerParams(use_tc_tiling_on_sc=False),
        scratch_types=[
            pltpu.MemorySpace.VMEM((cpw, C), jnp.int32),
            pltpu.MemorySpace.VMEM((cpw, C), jnp.int32),
            pltpu.MemorySpace.VMEM((cpw * C,), jnp.float32),
            pltpu.MemorySpace.VMEM((C, h), jnp.float32),
            pltpu.MemorySpace.VMEM_SHARED((n_nodes, h), jnp.float32),
            pltpu.SemaphoreType.DMA,
        ],
    )

    y1, dinv = pl.pallas_call(
        _tc_k1,
        out_shape=(
            jax.ShapeDtypeStruct((n_nodes, h), jnp.float32),
            jax.ShapeDtypeStruct((n_nodes,), jnp.float32),
        ),
    )(x, W1, degp)

    accp1 = agg_fn(src2d, dst2d, w, y1)

    y2 = pl.pallas_call(
        _tc_k2,
        out_shape=jax.ShapeDtypeStruct((n_nodes, h), jnp.float32),
    )(accp1, y1, dinv, b1, W2)

    accp2 = agg_fn(src2d, dst2d, w, y2)

    out = pl.pallas_call(
        _tc_k3,
        out_shape=jax.ShapeDtypeStruct((n_nodes,), jnp.float32),
    )(accp2, y2, dinv, b2, Wl, bl)

    return out


# speedup vs baseline: 47.5154x; 1.5195x over previous
import functools
import jax
import jax.numpy as jnp
import numpy as np
from jax import lax
from jax.experimental import pallas as pl
from jax.experimental.pallas import tpu as pltpu
from jax.experimental.pallas import tpu_sc as plsc

NC = 2
NS = 16
LANES = 16
C = 128


def _mesh():
    return plsc.VectorSubcoreMesh(
        core_axis_name="c", subcore_axis_name="s", num_cores=NC, num_subcores=NS
    )


def _deg_kernel(n_nodes, chunks_per_worker, dst_hbm, w_hbm, out_hbm,
                dst_v, w_v, acc_sh, zbuf):
    cid = lax.axis_index("c")
    sid = lax.axis_index("s")
    wid = sid * NC + cid
    start = wid * chunks_per_worker

    @pl.when(sid == 0)
    def _():
        def zero_body(i, _):
            zbuf[pl.ds(i * LANES, LANES)] = jnp.zeros((LANES,), jnp.float32)
            return _
        lax.fori_loop(0, n_nodes // LANES, zero_body, None)
        pltpu.sync_copy(zbuf, acc_sh)

    plsc.subcore_barrier()

    pltpu.sync_copy(dst_hbm.at[pl.ds(start, chunks_per_worker)], dst_v)
    pltpu.sync_copy(w_hbm.at[pl.ds(start, chunks_per_worker)], w_v)

    def chunk_body(j, _):
        pltpu.sync_copy(w_v.at[j], acc_sh.at[dst_v.at[j]], add=True)
        return _
    lax.fori_loop(0, chunks_per_worker, chunk_body, None)

    plsc.subcore_barrier()

    @pl.when(sid == 0)
    def _():
        pltpu.sync_copy(acc_sh, out_hbm.at[cid])


NBUF = 4


def _agg_kernel(n_nodes, chunks_per_worker, src_hbm, dst_hbm, w_hbm, y_hbm,
                out_hbm, src_v, dst_v, w_v, rows, gsems, ssems, acc_sh):
    cid = lax.axis_index("c")
    sid = lax.axis_index("s")
    wid = sid * NC + cid
    start = wid * chunks_per_worker
    cpw = chunks_per_worker

    @pl.when(sid == 0)
    def _():
        pltpu.sync_copy(y_hbm, acc_sh)

    plsc.subcore_barrier()

    pltpu.sync_copy(src_hbm.at[pl.ds(start, cpw)], src_v)
    pltpu.sync_copy(dst_hbm.at[pl.ds(start, cpw)], dst_v)
    pltpu.sync_copy(w_hbm.at[pl.ds(start * C, cpw * C)], w_v)

    def start_gather(j, b):
        pltpu.async_copy(y_hbm.at[src_v.at[j]], rows[b], gsems[b])

    def wait_gather(j, b):
        pltpu.make_async_copy(y_hbm.at[src_v.at[j]], rows[b], gsems[b]).wait()

    def start_scatter(j, b):
        pltpu.async_copy(rows[b], acc_sh.at[dst_v.at[j]], ssems[b], add=True)

    def wait_scatter(j, b):
        pltpu.make_async_copy(rows[b], acc_sh.at[dst_v.at[j]], ssems[b]).wait()

    def scale(j, b):
        base = j * C
        buf = rows[b]

        def scale_body(g, _):
            gbase = g * LANES
            wv = w_v[pl.ds(base + gbase, LANES)]
            for t in range(LANES):
                k = gbase + t
                wk = jnp.full((LANES,), wv[t])
                buf[k, pl.ds(0, LANES)] = buf[k, pl.ds(0, LANES)] * wk
                buf[k, pl.ds(LANES, LANES)] = buf[k, pl.ds(LANES, LANES)] * wk
            return _
        lax.fori_loop(0, C // LANES, scale_body, None)

    start_gather(0, 0)

    def ring_body(jo, _):
        for b in range(NBUF):
            j = jo * NBUF + b
            nb = (b + 1) % NBUF

            @pl.when(j >= NBUF - 1)
            def _():
                wait_scatter(j - (NBUF - 1), nb)

            @pl.when(j + 1 < cpw)
            def _():
                start_gather(j + 1, nb)

            wait_gather(j, b)
            scale(j, b)
            start_scatter(j, b)
        return _
    lax.fori_loop(0, cpw // NBUF, ring_body, None)

    for j in range(cpw - NBUF + 1, cpw):
        wait_scatter(j, j % NBUF)

    plsc.subcore_barrier()

    @pl.when(sid == 0)
    def _():
        pltpu.sync_copy(acc_sh, out_hbm.at[cid])


def _tc_k1(x_ref, w1_ref, degp_ref, y_ref, dinv_ref):
    deg = degp_ref[0, :] + degp_ref[1, :] + 1.0
    dinv = lax.rsqrt(deg)
    xw = jnp.dot(x_ref[...], w1_ref[...], preferred_element_type=jnp.float32)
    y_ref[...] = xw * dinv[:, None]
    dinv_ref[...] = dinv


def _tc_k2(accp_ref, y1_ref, dinv_ref, b1_ref, w2_ref, y2_ref):
    y1 = y1_ref[...]
    dinv = dinv_ref[...]
    t = accp_ref[0] + accp_ref[1] - y1
    h1 = jnp.maximum(dinv[:, None] * t + b1_ref[...][None, :], 0.0)
    xw2 = jnp.dot(h1, w2_ref[...], preferred_element_type=jnp.float32)
    y2_ref[...] = xw2 * dinv[:, None]


def _tc_k3(accp_ref, y2_ref, dinv_ref, b2_ref, wl_ref, bl_ref, out_ref):
    y2 = y2_ref[...]
    dinv = dinv_ref[...]
    t = accp_ref[0] + accp_ref[1] - y2
    h2 = jnp.maximum(dinv[:, None] * t + b2_ref[...][None, :], 0.0)
    res = jnp.dot(h2, wl_ref[...], preferred_element_type=jnp.float32)
    out_ref[...] = res[:, 0] + bl_ref[0]


@jax.jit
def kernel(x, edge_index, edge_weight, W1, b1, W2, b2, Wl, bl):
    n_nodes, d_in = x.shape
    h = W1.shape[1]
    e = edge_index.shape[1]

    n_workers = NC * NS
    chunks = -(-e // C)
    cpw = -(-chunks // n_workers)
    cpw = -(-cpw // 8) * 8
    e_pad = n_workers * cpw * C

    pad = e_pad - e
    src = edge_index[0]
    dst = edge_index[1]
    w = edge_weight
    if pad:
        fill = (jnp.arange(pad, dtype=jnp.int32) * 97) % n_nodes
        src = jnp.concatenate([src, fill])
        dst = jnp.concatenate([dst, fill])
        w = jnp.concatenate([w, jnp.zeros((pad,), jnp.float32)])
    src2d = src.reshape(-1, C)
    dst2d = dst.reshape(-1, C)
    w2d = w.reshape(-1, C)

    mesh = _mesh()

    deg_fn = pl.kernel(
        functools.partial(_deg_kernel, n_nodes, cpw),
        out_type=jax.ShapeDtypeStruct((NC, n_nodes), jnp.float32),
        mesh=mesh,
        scratch_types=[
            pltpu.MemorySpace.VMEM((cpw, C), jnp.int32),
            pltpu.MemorySpace.VMEM((cpw, C), jnp.float32),
            pltpu.MemorySpace.VMEM_SHARED((n_nodes,), jnp.float32),
            pltpu.MemorySpace.VMEM((n_nodes,), jnp.float32),
        ],
    )
    degp = deg_fn(dst2d, w2d)

    agg_fn = pl.kernel(
        functools.partial(_agg_kernel, n_nodes, cpw),
        out_type=jax.ShapeDtypeStruct((NC, n_nodes, h), jnp.float32),
        mesh=mesh,
        compiler_params=pltpu.CompilerParams(use_tc_tiling_on_sc=False),
        scratch_types=[
            pltpu.MemorySpace.VMEM((cpw, C), jnp.int32),
            pltpu.MemorySpace.VMEM((cpw, C), jnp.int32),
            pltpu.MemorySpace.VMEM((cpw * C,), jnp.float32),
            tuple(pltpu.MemorySpace.VMEM((C, h), jnp.float32) for _ in range(NBUF)),
            tuple(pltpu.SemaphoreType.DMA for _ in range(NBUF)),
            tuple(pltpu.SemaphoreType.DMA for _ in range(NBUF)),
            pltpu.MemorySpace.VMEM_SHARED((n_nodes, h), jnp.float32),
        ],
    )

    y1, dinv = pl.pallas_call(
        _tc_k1,
        out_shape=(
            jax.ShapeDtypeStruct((n_nodes, h), jnp.float32),
            jax.ShapeDtypeStruct((n_nodes,), jnp.float32),
        ),
    )(x, W1, degp)

    accp1 = agg_fn(src2d, dst2d, w, y1)

    y2 = pl.pallas_call(
        _tc_k2,
        out_shape=jax.ShapeDtypeStruct((n_nodes, h), jnp.float32),
    )(accp1, y1, dinv, b1, W2)

    accp2 = agg_fn(src2d, dst2d, w, y2)

    out = pl.pallas_call(
        _tc_k3,
        out_shape=jax.ShapeDtypeStruct((n_nodes,), jnp.float32),
    )(accp2, y2, dinv, b2, Wl, bl)

    return out


# speedup vs baseline: 52.0273x; 1.0950x over previous
import functools
import jax
import jax.numpy as jnp
import numpy as np
from jax import lax
from jax.experimental import pallas as pl
from jax.experimental.pallas import tpu as pltpu
from jax.experimental.pallas import tpu_sc as plsc

NC = 2
NS = 16
LANES = 16
C = 128


def _mesh():
    return plsc.VectorSubcoreMesh(
        core_axis_name="c", subcore_axis_name="s", num_cores=NC, num_subcores=NS
    )


def _deg_kernel(n_nodes, chunks_per_worker, dst_hbm, w_hbm, out_hbm,
                dst_v, w_v, ssems, acc_sh, zbuf):
    cid = lax.axis_index("c")
    sid = lax.axis_index("s")
    wid = sid * NC + cid
    start = wid * chunks_per_worker
    cpw = chunks_per_worker

    @pl.when(sid == 0)
    def _():
        def zero_body(i, _):
            zbuf[pl.ds(i * LANES, LANES)] = jnp.zeros((LANES,), jnp.float32)
            return _
        lax.fori_loop(0, n_nodes // LANES, zero_body, None)
        pltpu.sync_copy(zbuf, acc_sh)

    plsc.subcore_barrier()

    pltpu.sync_copy(dst_hbm.at[pl.ds(start, cpw)], dst_v)
    pltpu.sync_copy(w_hbm.at[pl.ds(start, cpw)], w_v)

    def ring_body(jo, _):
        for b in range(NBUF):
            j = jo * NBUF + b

            @pl.when(j >= NBUF)
            def _():
                pltpu.make_async_copy(
                    w_v.at[j - NBUF], acc_sh.at[dst_v.at[j - NBUF]], ssems[b]
                ).wait()

            pltpu.async_copy(w_v.at[j], acc_sh.at[dst_v.at[j]], ssems[b],
                             add=True)
        return _
    lax.fori_loop(0, cpw // NBUF, ring_body, None)

    for j in range(cpw - NBUF, cpw):
        pltpu.make_async_copy(
            w_v.at[j], acc_sh.at[dst_v.at[j]], ssems[j % NBUF]
        ).wait()

    plsc.subcore_barrier()

    @pl.when(sid == 0)
    def _():
        pltpu.sync_copy(acc_sh, out_hbm.at[cid])


NBUF = 8


def _agg_kernel(n_nodes, chunks_per_worker, src_hbm, dst_hbm, w_hbm, y_hbm,
                out_hbm, src_v, dst_v, w_v, rows, gsems, ssems, acc_sh, y_sh):
    cid = lax.axis_index("c")
    sid = lax.axis_index("s")
    wid = sid * NC + cid
    start = wid * chunks_per_worker
    cpw = chunks_per_worker

    @pl.when(sid == 0)
    def _():
        pltpu.sync_copy(y_hbm, acc_sh)

    @pl.when(sid == 1)
    def _():
        pltpu.sync_copy(y_hbm, y_sh)

    plsc.subcore_barrier()

    pltpu.sync_copy(src_hbm.at[pl.ds(start, cpw)], src_v)
    pltpu.sync_copy(dst_hbm.at[pl.ds(start, cpw)], dst_v)
    pltpu.sync_copy(w_hbm.at[pl.ds(start * C, cpw * C)], w_v)

    def start_gather(j, b):
        pltpu.async_copy(y_sh.at[src_v.at[j]], rows[b], gsems[b])

    def wait_gather(j, b):
        pltpu.make_async_copy(y_sh.at[src_v.at[j]], rows[b], gsems[b]).wait()

    def start_scatter(j, b):
        pltpu.async_copy(rows[b], acc_sh.at[dst_v.at[j]], ssems[b], add=True)

    def wait_scatter(j, b):
        pltpu.make_async_copy(rows[b], acc_sh.at[dst_v.at[j]], ssems[b]).wait()

    def scale(j, b):
        base = j * C
        buf = rows[b]

        def scale_body(g, _):
            gbase = g * LANES
            wv = w_v[pl.ds(base + gbase, LANES)]
            for t in range(LANES):
                k = gbase + t
                wk = jnp.full((LANES,), wv[t])
                buf[k, pl.ds(0, LANES)] = buf[k, pl.ds(0, LANES)] * wk
                buf[k, pl.ds(LANES, LANES)] = buf[k, pl.ds(LANES, LANES)] * wk
            return _
        lax.fori_loop(0, C // LANES, scale_body, None)

    start_gather(0, 0)

    def ring_body(jo, _):
        for b in range(NBUF):
            j = jo * NBUF + b
            nb = (b + 1) % NBUF

            @pl.when(j >= NBUF - 1)
            def _():
                wait_scatter(j - (NBUF - 1), nb)

            @pl.when(j + 1 < cpw)
            def _():
                start_gather(j + 1, nb)

            wait_gather(j, b)
            scale(j, b)
            start_scatter(j, b)
        return _
    lax.fori_loop(0, cpw // NBUF, ring_body, None)

    for j in range(cpw - NBUF + 1, cpw):
        wait_scatter(j, j % NBUF)

    plsc.subcore_barrier()

    @pl.when(sid == 0)
    def _():
        pltpu.sync_copy(acc_sh, out_hbm.at[cid])


def _tc_k1(x_ref, w1_ref, degp_ref, y_ref, dinv_ref):
    deg = degp_ref[0, :] + degp_ref[1, :] + 1.0
    dinv = lax.rsqrt(deg)
    xw = jnp.dot(x_ref[...], w1_ref[...], preferred_element_type=jnp.float32)
    y_ref[...] = xw * dinv[:, None]
    dinv_ref[...] = dinv


def _tc_k2(accp_ref, y1_ref, dinv_ref, b1_ref, w2_ref, y2_ref):
    y1 = y1_ref[...]
    dinv = dinv_ref[...]
    t = accp_ref[0] + accp_ref[1] - y1
    h1 = jnp.maximum(dinv[:, None] * t + b1_ref[...][None, :], 0.0)
    xw2 = jnp.dot(h1, w2_ref[...], preferred_element_type=jnp.float32)
    y2_ref[...] = xw2 * dinv[:, None]


def _tc_k3(accp_ref, y2_ref, dinv_ref, b2_ref, wl_ref, bl_ref, out_ref):
    y2 = y2_ref[...]
    dinv = dinv_ref[...]
    t = accp_ref[0] + accp_ref[1] - y2
    h2 = jnp.maximum(dinv[:, None] * t + b2_ref[...][None, :], 0.0)
    res = jnp.dot(h2, wl_ref[...], preferred_element_type=jnp.float32)
    out_ref[...] = res[:, 0] + bl_ref[0]


@jax.jit
def kernel(x, edge_index, edge_weight, W1, b1, W2, b2, Wl, bl):
    n_nodes, d_in = x.shape
    h = W1.shape[1]
    e = edge_index.shape[1]

    n_workers = NC * NS
    chunks = -(-e // C)
    cpw = -(-chunks // n_workers)
    cpw = -(-cpw // 8) * 8
    e_pad = n_workers * cpw * C

    pad = e_pad - e
    src = edge_index[0]
    dst = edge_index[1]
    w = edge_weight
    if pad:
        fill = (jnp.arange(pad, dtype=jnp.int32) * 97) % n_nodes
        src = jnp.concatenate([src, fill])
        dst = jnp.concatenate([dst, fill])
        w = jnp.concatenate([w, jnp.zeros((pad,), jnp.float32)])
    src2d = src.reshape(-1, C)
    dst2d = dst.reshape(-1, C)
    w2d = w.reshape(-1, C)

    mesh = _mesh()

    deg_fn = pl.kernel(
        functools.partial(_deg_kernel, n_nodes, cpw),
        out_type=jax.ShapeDtypeStruct((NC, n_nodes), jnp.float32),
        mesh=mesh,
        scratch_types=[
            pltpu.MemorySpace.VMEM((cpw, C), jnp.int32),
            pltpu.MemorySpace.VMEM((cpw, C), jnp.float32),
            tuple(pltpu.SemaphoreType.DMA for _ in range(NBUF)),
            pltpu.MemorySpace.VMEM_SHARED((n_nodes,), jnp.float32),
            pltpu.MemorySpace.VMEM((n_nodes,), jnp.float32),
        ],
    )
    degp = deg_fn(dst2d, w2d)

    agg_fn = pl.kernel(
        functools.partial(_agg_kernel, n_nodes, cpw),
        out_type=jax.ShapeDtypeStruct((NC, n_nodes, h), jnp.float32),
        mesh=mesh,
        compiler_params=pltpu.CompilerParams(use_tc_tiling_on_sc=False),
        scratch_types=[
            pltpu.MemorySpace.VMEM((cpw, C), jnp.int32),
            pltpu.MemorySpace.VMEM((cpw, C), jnp.int32),
            pltpu.MemorySpace.VMEM((cpw * C,), jnp.float32),
            tuple(pltpu.MemorySpace.VMEM((C, h), jnp.float32) for _ in range(NBUF)),
            tuple(pltpu.SemaphoreType.DMA for _ in range(NBUF)),
            tuple(pltpu.SemaphoreType.DMA for _ in range(NBUF)),
            pltpu.MemorySpace.VMEM_SHARED((n_nodes, h), jnp.float32),
            pltpu.MemorySpace.VMEM_SHARED((n_nodes, h), jnp.float32),
        ],
    )

    y1, dinv = pl.pallas_call(
        _tc_k1,
        out_shape=(
            jax.ShapeDtypeStruct((n_nodes, h), jnp.float32),
            jax.ShapeDtypeStruct((n_nodes,), jnp.float32),
        ),
    )(x, W1, degp)

    accp1 = agg_fn(src2d, dst2d, w, y1)

    y2 = pl.pallas_call(
        _tc_k2,
        out_shape=jax.ShapeDtypeStruct((n_nodes, h), jnp.float32),
    )(accp1, y1, dinv, b1, W2)

    accp2 = agg_fn(src2d, dst2d, w, y2)

    out = pl.pallas_call(
        _tc_k3,
        out_shape=jax.ShapeDtypeStruct((n_nodes,), jnp.float32),
    )(accp2, y2, dinv, b2, Wl, bl)

    return out
